# Initial kernel scaffold; baseline (speedup 1.0000x reference)
import jax
import jax.numpy as jnp
from jax.experimental import pallas as pl


def kernel(x, table):
    raise NotImplementedError("write your pallas kernel here")


# speedup vs baseline: 4.5901x; 4.5901x over previous
import functools

import jax
import jax.numpy as jnp
from jax import lax
from jax.experimental import pallas as pl
from jax.experimental.pallas import tpu as pltpu
from jax.experimental.pallas import tpu_sc as plsc


def _build_emb_kernel(N, D, n_per_w, C, n_chunks, num_cores):
    mesh = plsc.VectorSubcoreMesh(core_axis_name="c", subcore_axis_name="s")

    @functools.partial(
        pl.kernel,
        mesh=mesh,
        out_type=jax.ShapeDtypeStruct((N, D), jnp.float32),
        compiler_params=pltpu.CompilerParams(use_tc_tiling_on_sc=False),
        scratch_types=[
            pltpu.VMEM((n_per_w,), jnp.int32),
            pltpu.VMEM((C, D), jnp.float32),
            pltpu.SemaphoreType.DMA,
        ],
    )
    def emb_kernel(idx_hbm, table_hbm, out_hbm, idx_v, buf, sem):
        wid = lax.axis_index("s") * num_cores + lax.axis_index("c")
        base = wid * n_per_w
        pltpu.sync_copy(idx_hbm.at[pl.ds(base, n_per_w)], idx_v)

        def body(c, carry):
            off = pl.multiple_of(c * C, 8)
            pltpu.async_copy(table_hbm.at[idx_v.at[pl.ds(off, C)]], buf, sem).wait()
            pltpu.sync_copy(buf, out_hbm.at[pl.ds(base + off, C)])
            return carry

        lax.fori_loop(0, n_chunks, body, 0)

    return emb_kernel


def kernel(x, table):
    B, H = x.shape
    V, D = table.shape
    N = B * H

    info = plsc.get_sparse_core_info()
    NW = info.num_cores * info.num_subcores

    n_per_w = N // NW
    C = 800
    n_chunks = n_per_w // C

    idx_flat = x.reshape(N).astype(jnp.int32)
    out = _build_emb_kernel(N, D, n_per_w, C, n_chunks, info.num_cores)(
        idx_flat, table)
    return out.reshape(B, H, D)


# speedup vs baseline: 4.6568x; 1.0145x over previous
import functools

import jax
import jax.numpy as jnp
from jax import lax
from jax.experimental import pallas as pl
from jax.experimental.pallas import tpu as pltpu
from jax.experimental.pallas import tpu_sc as plsc


def _build_emb_kernel(N, D, n_per_w, C, n_chunks, num_cores):
    mesh = plsc.VectorSubcoreMesh(core_axis_name="c", subcore_axis_name="s")

    @functools.partial(
        pl.kernel,
        mesh=mesh,
        out_type=jax.ShapeDtypeStruct((N, D), jnp.float32),
        compiler_params=pltpu.CompilerParams(use_tc_tiling_on_sc=False),
        scratch_types=[
            pltpu.VMEM((n_per_w,), jnp.int32),
            pltpu.VMEM((C, D), jnp.float32),
            pltpu.VMEM((C, D), jnp.float32),
            pltpu.SemaphoreType.DMA,
            pltpu.SemaphoreType.DMA,
            pltpu.SemaphoreType.DMA,
            pltpu.SemaphoreType.DMA,
        ],
    )
    def emb_kernel(idx_hbm, table_hbm, out_hbm, idx_v, buf0, buf1,
                   gsem0, gsem1, ssem0, ssem1):
        wid = lax.axis_index("s") * num_cores + lax.axis_index("c")
        base = wid * n_per_w
        pltpu.sync_copy(idx_hbm.at[pl.ds(base, n_per_w)], idx_v)

        bufs = (buf0, buf1)
        gsems = (gsem0, gsem1)
        ssems = (ssem0, ssem1)

        def gather(c, b):
            return pltpu.async_copy(
                table_hbm.at[idx_v.at[pl.ds(c * C, C)]], bufs[b], gsems[b])

        def store(c, b):
            return pltpu.async_copy(
                bufs[b], out_hbm.at[pl.ds(base + c * C, C)], ssems[b])

        gh = [None, None]
        sh = [None, None]
        gh[0] = gather(0, 0)
        for c in range(n_chunks):
            b = c % 2
            nb = (c + 1) % 2
            if c + 1 < n_chunks:
                if sh[nb] is not None:
                    sh[nb].wait()
                gh[nb] = gather(c + 1, nb)
            gh[b].wait()
            sh[b] = store(c, b)
        for h in sh:
            if h is not None:
                h.wait()

    return emb_kernel


def kernel(x, table):
    B, H = x.shape
    V, D = table.shape
    N = B * H

    info = plsc.get_sparse_core_info()
    NW = info.num_cores * info.num_subcores

    n_per_w = N // NW
    C = 800
    n_chunks = n_per_w // C

    idx_flat = x.reshape(N).astype(jnp.int32)
    out = _build_emb_kernel(N, D, n_per_w, C, n_chunks, info.num_cores)(
        idx_flat, table)
    return out.reshape(B, H, D)
